# baseline (speedup 1.0000x reference)
import functools

import jax
import jax.numpy as jnp
import numpy as np
from jax import lax
from jax.experimental import pallas as pl
from jax.experimental.pallas import tpu as pltpu
from jax.experimental.pallas import tpu_sc as plsc

_MASK_FRAC = 0.5
_S = 16
_H = 224
_SCALE = _H // _S
_NP = 768
_NC = 2
_NS = 16
_NW = _NC * _NS
_PPW = _NP // _NW
_VL = 16
_VPR = _H // _VL
_CH = 56

_SEGS = {
    0: [("c", 0, 7, 0, 0), ("l", 7, 14, 0, 0), ("l", 21, 14, 1, 0),
        ("l", 35, 14, 2, 0), ("l", 49, 7, 3, 0)],
    1: [("l", 0, 7, 3, 7), ("l", 7, 14, 4, 0), ("l", 21, 14, 5, 0),
        ("l", 35, 14, 6, 0), ("l", 49, 7, 7, 0)],
    2: [("l", 0, 7, 7, 7), ("l", 7, 14, 8, 0), ("l", 21, 14, 9, 0),
        ("l", 35, 14, 10, 0), ("l", 49, 7, 11, 0)],
    3: [("l", 0, 7, 11, 7), ("l", 7, 14, 12, 0), ("l", 21, 14, 13, 0),
        ("l", 35, 14, 14, 0), ("c", 49, 7, 15, 0)],
}


def _host_tables():
    s = (np.arange(_H) + 0.5) / _SCALE - 0.5
    f = np.floor(s)
    w = (s - f).astype(np.float32)
    i0 = np.clip(f, 0, _S - 1).astype(np.int32)
    i1 = np.clip(f + 1, 0, _S - 1).astype(np.int32)
    return i0, i1, (1.0 - w), w


def _sc_body(x_hbm, mask_hbm, i0_hbm, i1_hbm, w0_hbm, w1_hbm, out_hbm,
             b0, b1, b2, b3, mh, mv, i0v, i1v, w0v, w1v,
             si0, si1, si2, si3, so0, so1, so2, so3):
    bufs = [b0, b1, b2, b3]
    sin = [si0, si1, si2, si3]
    sout = [so0, so1, so2, so3]
    wid = lax.axis_index("s") * _NC + lax.axis_index("c")
    base_p = wid * _PPW

    pltpu.sync_copy(i0_hbm, i0v)
    pltpu.sync_copy(i1_hbm, i1v)
    pltpu.sync_copy(w0_hbm, w0v)
    pltpu.sync_copy(w1_hbm, w1v)

    def in_slice(p, cp):
        return x_hbm.at[p, pl.ds(cp * _CH, _CH)]

    def out_slice(p, cp):
        return out_hbm.at[p, pl.ds(cp * _CH, _CH)]

    zero = jnp.zeros((_VL,), jnp.float32)

    pltpu.sync_copy(mask_hbm.at[wid], mv)

    i0s = [i0v[pl.ds(v * _VL, _VL)] for v in range(_VPR)]
    i1s = [i1v[pl.ds(v * _VL, _VL)] for v in range(_VPR)]
    w0s = [w0v[pl.ds(v * _VL, _VL)] for v in range(_VPR)]
    w1s = [w1v[pl.ds(v * _VL, _VL)] for v in range(_VPR)]

    def build_mh(j):
        mbase = j * (_S * _S)

        def r_body(r, c):
            ro = mbase + r * _S
            for v in range(_VPR):
                g0 = plsc.load_gather(mv, [i0s[v] + ro])
                g1 = plsc.load_gather(mv, [i1s[v] + ro])
                mh[pl.ds(r * _H + v * _VL, _VL)] = g0 * w0s[v] + g1 * w1s[v]
            return c
        lax.fori_loop(0, _S, r_body, 0)

    cols = [lax.iota(jnp.int32, _VL) + v * _VL for v in range(_VPR)]

    def seg_const(buf, r0, n, mrow):
        sel = [mh[pl.ds(mrow * _H + v * _VL, _VL)] < _MASK_FRAC
               for v in range(_VPR)]

        def one_row(r):
            rv = jnp.full((_VL,), 0, jnp.int32) + r
            for v in range(_VPR):
                plsc.store_scatter(buf, [rv, cols[v]], zero, mask=sel[v])

        def rb(i, c):
            r = r0 + 2 * i
            one_row(r)
            one_row(r + 1)
            return c
        lax.fori_loop(0, n // 2, rb, 0)
        if n % 2:
            one_row(r0 + n - 1)

    def seg_lerp(buf, r0, n, mrow, woff):
        m0s = [mh[pl.ds(mrow * _H + v * _VL, _VL)] for v in range(_VPR)]
        dvs = [mh[pl.ds((mrow + 1) * _H + v * _VL, _VL)] - m0s[v]
               for v in range(_VPR)]

        def one_row(r, wy):
            wyv = jnp.full((_VL,), 0.0, jnp.float32) + wy
            rv = jnp.full((_VL,), 0, jnp.int32) + r
            for v in range(_VPR):
                m = m0s[v] + wyv * dvs[v]
                plsc.store_scatter(buf, [rv, cols[v]], zero,
                                   mask=m < _MASK_FRAC)

        def rb(i, c):
            i2 = 2 * i
            wy = (i2.astype(jnp.float32) + (woff + 0.5)) * (1.0 / _SCALE)
            r = r0 + i2
            one_row(r, wy)
            one_row(r + 1, wy + 1.0 / _SCALE)
            return c
        lax.fori_loop(0, n // 2, rb, 0)
        if n % 2:
            one_row(r0 + n - 1, (n - 1 + woff + 0.5) * (1.0 / _SCALE))

    pltpu.async_copy(in_slice(base_p, 0), bufs[0], sin[0])
    pltpu.async_copy(in_slice(base_p, 1), bufs[1], sin[1])

    def body(j, carry):
        p = base_p + j
        for i in range(4):
            bn = (i + 2) % 4

            def drain():
                pltpu.make_async_copy(bufs[bn], out_slice(p, 0),
                                      sout[bn]).wait()
            if i < 2:
                pl.when(j > 0)(drain)
            else:
                drain()

            if i < 2:
                pltpu.async_copy(in_slice(p, i + 2), bufs[bn], sin[bn])
            else:
                def refill():
                    pltpu.async_copy(in_slice(p + 1, i - 2), bufs[bn],
                                     sin[bn])
                pl.when(j < _PPW - 1)(refill)

            pltpu.make_async_copy(in_slice(p, i), bufs[i], sin[i]).wait()

            if i == 0:
                build_mh(j)

            for kind, r0, n, mrow, woff in _SEGS[i]:
                if kind == "c":
                    seg_const(bufs[i], r0, n, mrow)
                else:
                    seg_lerp(bufs[i], r0, n, mrow, woff)

            pltpu.async_copy(bufs[i], out_slice(p, i), sout[i])
        return carry

    lax.fori_loop(0, _PPW, body, 0)

    last = base_p + _PPW - 1
    pltpu.make_async_copy(bufs[2], out_slice(last, 2), sout[2]).wait()
    pltpu.make_async_copy(bufs[3], out_slice(last, 3), sout[3]).wait()


@jax.jit
def _run(x, mask):
    B, C, H, W = x.shape
    xp = x.reshape(_NP, _H, _H)
    mp = mask.reshape(_NW, _PPW * _S * _S)
    i0, i1, w0, w1 = _host_tables()

    mesh = plsc.VectorSubcoreMesh(core_axis_name="c", subcore_axis_name="s",
                                  num_cores=_NC, num_subcores=_NS)
    fn = functools.partial(
        pl.kernel,
        out_type=jax.ShapeDtypeStruct((_NP, _H, _H), jnp.float32),
        mesh=mesh,
        compiler_params=pltpu.CompilerParams(needs_layout_passes=False),
        scratch_types=[
            pltpu.VMEM((_CH, _H), jnp.float32),
            pltpu.VMEM((_CH, _H), jnp.float32),
            pltpu.VMEM((_CH, _H), jnp.float32),
            pltpu.VMEM((_CH, _H), jnp.float32),
            pltpu.VMEM((_S * _H,), jnp.float32),
            pltpu.VMEM((_PPW * _S * _S,), jnp.float32),
            pltpu.VMEM((_H,), jnp.int32),
            pltpu.VMEM((_H,), jnp.int32),
            pltpu.VMEM((_H,), jnp.float32),
            pltpu.VMEM((_H,), jnp.float32),
        ] + [pltpu.SemaphoreType.DMA] * 8,
    )(_sc_body)
    out = fn(xp, mp, jnp.asarray(i0), jnp.asarray(i1),
             jnp.asarray(w0), jnp.asarray(w1))
    return out.reshape(B, C, H, W)


def kernel(x, mask):
    return _run(x, mask)


# speedup vs baseline: 1.9780x; 1.9780x over previous
import functools

import jax
import jax.numpy as jnp
import numpy as np
from jax import lax
from jax.experimental import pallas as pl
from jax.experimental.pallas import tpu as pltpu
from jax.experimental.pallas import tpu_sc as plsc

_MASK_FRAC = 0.5
_S = 16
_H = 224
_SCALE = _H // _S
_NP = 768
_NC = 2
_NS = 16
_NW = _NC * _NS
_PPW = _NP // _NW
_VL = 16
_VPR = _H // _VL
_CH = 112

_SEGS = {
    0: [("c", 0, 7, 0, 0), ("l", 7, 14, 0, 0), ("l", 21, 14, 1, 0),
        ("l", 35, 14, 2, 0), ("l", 49, 14, 3, 0), ("l", 63, 14, 4, 0),
        ("l", 77, 14, 5, 0), ("l", 91, 14, 6, 0), ("l", 105, 7, 7, 0)],
    1: [("l", 0, 7, 7, 7), ("l", 7, 14, 8, 0), ("l", 21, 14, 9, 0),
        ("l", 35, 14, 10, 0), ("l", 49, 14, 11, 0), ("l", 63, 14, 12, 0),
        ("l", 77, 14, 13, 0), ("l", 91, 14, 14, 0), ("c", 105, 7, 15, 0)],
}


def _host_tables():
    s = (np.arange(_H) + 0.5) / _SCALE - 0.5
    f = np.floor(s)
    w = (s - f).astype(np.float32)
    i0 = np.clip(f, 0, _S - 1).astype(np.int32)
    i1 = np.clip(f + 1, 0, _S - 1).astype(np.int32)
    return i0, i1, (1.0 - w), w


def _sc_body(x_hbm, mask_hbm, i0_hbm, i1_hbm, w0_hbm, w1_hbm, out_hbm,
             b0, b1, b2, b3, mh, mv, i0v, i1v, w0v, w1v,
             si0, si1, si2, si3, so0, so1, so2, so3):
    bufs = [b0, b1, b2, b3]
    sin = [si0, si1, si2, si3]
    sout = [so0, so1, so2, so3]
    wid = lax.axis_index("s") * _NC + lax.axis_index("c")
    base_p = wid * _PPW

    pltpu.sync_copy(i0_hbm, i0v)
    pltpu.sync_copy(i1_hbm, i1v)
    pltpu.sync_copy(w0_hbm, w0v)
    pltpu.sync_copy(w1_hbm, w1v)

    def in_slice(p, cp):
        return x_hbm.at[p, pl.ds(cp * _CH, _CH)]

    def out_slice(p, cp):
        return out_hbm.at[p, pl.ds(cp * _CH, _CH)]

    zero = jnp.zeros((_VL,), jnp.float32)

    pltpu.sync_copy(mask_hbm.at[wid], mv)

    i0s = [i0v[pl.ds(v * _VL, _VL)] for v in range(_VPR)]
    i1s = [i1v[pl.ds(v * _VL, _VL)] for v in range(_VPR)]
    w0s = [w0v[pl.ds(v * _VL, _VL)] for v in range(_VPR)]
    w1s = [w1v[pl.ds(v * _VL, _VL)] for v in range(_VPR)]

    def build_mh(j):
        mbase = j * (_S * _S)

        def r_body(r, c):
            ro = mbase + r * _S
            for v in range(_VPR):
                g0 = plsc.load_gather(mv, [i0s[v] + ro])
                g1 = plsc.load_gather(mv, [i1s[v] + ro])
                mh[pl.ds(r * _H + v * _VL, _VL)] = g0 * w0s[v] + g1 * w1s[v]
            return c
        lax.fori_loop(0, _S, r_body, 0)

    def seg_const(buf, r0, n, mrow):
        sel = [mh[pl.ds(mrow * _H + v * _VL, _VL)] < _MASK_FRAC
               for v in range(_VPR)]

        def one_row(r):
            for v in range(_VPR):
                sl = pl.ds(v * _VL, _VL)
                buf[r, sl] = jnp.where(sel[v], zero, buf[r, sl])

        def rb(i, c):
            r = r0 + 2 * i
            one_row(r)
            one_row(r + 1)
            return c
        lax.fori_loop(0, n // 2, rb, 0)
        if n % 2:
            one_row(r0 + n - 1)

    def seg_lerp(buf, r0, n, mrow, woff):
        m0s = [mh[pl.ds(mrow * _H + v * _VL, _VL)] for v in range(_VPR)]
        dvs = [mh[pl.ds((mrow + 1) * _H + v * _VL, _VL)] - m0s[v]
               for v in range(_VPR)]

        def one_row(r, wy):
            wyv = jnp.full((_VL,), 0.0, jnp.float32) + wy
            for v in range(_VPR):
                sl = pl.ds(v * _VL, _VL)
                m = m0s[v] + wyv * dvs[v]
                buf[r, sl] = jnp.where(m < _MASK_FRAC, zero, buf[r, sl])

        def rb(i, c):
            i2 = 2 * i
            wy = (i2.astype(jnp.float32) + (woff + 0.5)) * (1.0 / _SCALE)
            r = r0 + i2
            one_row(r, wy)
            one_row(r + 1, wy + 1.0 / _SCALE)
            return c
        lax.fori_loop(0, n // 2, rb, 0)
        if n % 2:
            one_row(r0 + n - 1, (n - 1 + woff + 0.5) * (1.0 / _SCALE))

    pltpu.async_copy(in_slice(base_p, 0), bufs[0], sin[0])
    pltpu.async_copy(in_slice(base_p, 1), bufs[1], sin[1])

    def body(j, carry):
        for i in range(4):
            bn = (i + 2) % 4
            pll = 2 * j + i // 2
            cp = i % 2

            def drain():
                pltpu.make_async_copy(bufs[bn], out_slice(base_p, 0),
                                      sout[bn]).wait()
            if i < 2:
                pl.when(j > 0)(drain)
            else:
                drain()

            def refill():
                pltpu.async_copy(in_slice(base_p + pll + 1, cp), bufs[bn],
                                 sin[bn])
            if i < 2:
                refill()
            else:
                pl.when(j < _PPW // 2 - 1)(refill)

            pltpu.make_async_copy(in_slice(base_p + pll, cp),
                                  bufs[i], sin[i]).wait()

            if cp == 0:
                build_mh(pll)

            for kind, r0, n, mrow, woff in _SEGS[cp]:
                if kind == "c":
                    seg_const(bufs[i], r0, n, mrow)
                else:
                    seg_lerp(bufs[i], r0, n, mrow, woff)

            pltpu.async_copy(bufs[i], out_slice(base_p + pll, cp), sout[i])
        return carry

    lax.fori_loop(0, _PPW // 2, body, 0)

    last = base_p + _PPW - 1
    pltpu.make_async_copy(bufs[2], out_slice(last, 0), sout[2]).wait()
    pltpu.make_async_copy(bufs[3], out_slice(last, 1), sout[3]).wait()


@jax.jit
def _run(x, mask):
    B, C, H, W = x.shape
    xp = x.reshape(_NP, _H, _H)
    mp = mask.reshape(_NW, _PPW * _S * _S)
    i0, i1, w0, w1 = _host_tables()

    mesh = plsc.VectorSubcoreMesh(core_axis_name="c", subcore_axis_name="s",
                                  num_cores=_NC, num_subcores=_NS)
    fn = functools.partial(
        pl.kernel,
        out_type=jax.ShapeDtypeStruct((_NP, _H, _H), jnp.float32),
        mesh=mesh,
        compiler_params=pltpu.CompilerParams(needs_layout_passes=False),
        scratch_types=[
            pltpu.VMEM((_CH, _H), jnp.float32),
            pltpu.VMEM((_CH, _H), jnp.float32),
            pltpu.VMEM((_CH, _H), jnp.float32),
            pltpu.VMEM((_CH, _H), jnp.float32),
            pltpu.VMEM((_S * _H,), jnp.float32),
            pltpu.VMEM((_PPW * _S * _S,), jnp.float32),
            pltpu.VMEM((_H,), jnp.int32),
            pltpu.VMEM((_H,), jnp.int32),
            pltpu.VMEM((_H,), jnp.float32),
            pltpu.VMEM((_H,), jnp.float32),
        ] + [pltpu.SemaphoreType.DMA] * 8,
    )(_sc_body)
    out = fn(xp, mp, jnp.asarray(i0), jnp.asarray(i1),
             jnp.asarray(w0), jnp.asarray(w1))
    return out.reshape(B, C, H, W)


def kernel(x, mask):
    return _run(x, mask)


# speedup vs baseline: 2.0385x; 1.0306x over previous
import functools

import jax
import jax.numpy as jnp
import numpy as np
from jax import lax
from jax.experimental import pallas as pl
from jax.experimental.pallas import tpu as pltpu
from jax.experimental.pallas import tpu_sc as plsc

_MASK_FRAC = 0.5
_S = 16
_H = 224
_SCALE = _H // _S
_NP = 768
_NC = 2
_NS = 16
_NW = _NC * _NS
_PPW = _NP // _NW
_VL = 16
_VPR = _H // _VL
_CH = 56

_SEGS = {
    0: [("c", 0, 7, 0, 0), ("l", 7, 14, 0, 0), ("l", 21, 14, 1, 0),
        ("l", 35, 14, 2, 0), ("l", 49, 7, 3, 0)],
    1: [("l", 0, 7, 3, 7), ("l", 7, 14, 4, 0), ("l", 21, 14, 5, 0),
        ("l", 35, 14, 6, 0), ("l", 49, 7, 7, 0)],
    2: [("l", 0, 7, 7, 7), ("l", 7, 14, 8, 0), ("l", 21, 14, 9, 0),
        ("l", 35, 14, 10, 0), ("l", 49, 7, 11, 0)],
    3: [("l", 0, 7, 11, 7), ("l", 7, 14, 12, 0), ("l", 21, 14, 13, 0),
        ("l", 35, 14, 14, 0), ("c", 49, 7, 15, 0)],
}


def _host_tables():
    s = (np.arange(_H) + 0.5) / _SCALE - 0.5
    f = np.floor(s)
    w = (s - f).astype(np.float32)
    i0 = np.clip(f, 0, _S - 1).astype(np.int32)
    i1 = np.clip(f + 1, 0, _S - 1).astype(np.int32)
    return i0, i1, (1.0 - w), w


def _sc_body(x_hbm, mask_hbm, i0_hbm, w1_hbm, out_hbm,
             b0, b1, b2, b3, mh, mv, i0v, w1v,
             si0, si1, si2, si3, so0, so1, so2, so3):
    bufs = [b0, b1, b2, b3]
    sin = [si0, si1, si2, si3]
    sout = [so0, so1, so2, so3]
    wid = lax.axis_index("s") * _NC + lax.axis_index("c")
    base_p = wid * _PPW

    def in_slice(p, cp):
        return x_hbm.at[p, pl.ds(cp * _CH, _CH)]

    def out_slice(p, cp):
        return out_hbm.at[p, pl.ds(cp * _CH, _CH)]

    zero = jnp.zeros((_VL,), jnp.float32)

    pltpu.async_copy(in_slice(base_p, 0), bufs[0], sin[0])
    pltpu.async_copy(in_slice(base_p, 1), bufs[1], sin[1])

    pltpu.sync_copy(i0_hbm, i0v)
    pltpu.sync_copy(w1_hbm, w1v)
    pltpu.sync_copy(mask_hbm.at[wid], mv)

    i0s = [i0v[pl.ds(v * _VL, _VL)] for v in range(_VPR)]
    i1s = [jnp.minimum(i0s[v] + 1, _S - 1) for v in range(_VPR)]
    w1s = [w1v[pl.ds(v * _VL, _VL)] for v in range(_VPR)]
    w0s = [1.0 - w1s[v] for v in range(_VPR)]

    def build_mh(j):
        mbase = j * (_S * _S)

        def r_body(r, c):
            ro = mbase + r * _S
            for v in range(_VPR):
                g0 = plsc.load_gather(mv, [i0s[v] + ro])
                g1 = plsc.load_gather(mv, [i1s[v] + ro])
                mh[pl.ds(r * _H + v * _VL, _VL)] = g0 * w0s[v] + g1 * w1s[v]
            return c
        lax.fori_loop(0, _S, r_body, 0)

    def seg_const(buf, r0, n, mrow):
        sel = [mh[pl.ds(mrow * _H + v * _VL, _VL)] < _MASK_FRAC
               for v in range(_VPR)]

        def one_row(r):
            for v in range(_VPR):
                sl = pl.ds(v * _VL, _VL)
                buf[r, sl] = jnp.where(sel[v], zero, buf[r, sl])

        def rb(i, c):
            r = r0 + 2 * i
            one_row(r)
            one_row(r + 1)
            return c
        lax.fori_loop(0, n // 2, rb, 0)
        if n % 2:
            one_row(r0 + n - 1)

    def seg_lerp(buf, r0, n, mrow, woff):
        m0s = [mh[pl.ds(mrow * _H + v * _VL, _VL)] for v in range(_VPR)]
        dvs = [mh[pl.ds((mrow + 1) * _H + v * _VL, _VL)] - m0s[v]
               for v in range(_VPR)]

        def one_row(r, wy):
            wyv = jnp.full((_VL,), 0.0, jnp.float32) + wy
            for v in range(_VPR):
                sl = pl.ds(v * _VL, _VL)
                m = m0s[v] + wyv * dvs[v]
                buf[r, sl] = jnp.where(m < _MASK_FRAC, zero, buf[r, sl])

        def rb(i, c):
            i2 = 2 * i
            wy = (i2.astype(jnp.float32) + (woff + 0.5)) * (1.0 / _SCALE)
            r = r0 + i2
            one_row(r, wy)
            one_row(r + 1, wy + 1.0 / _SCALE)
            return c
        lax.fori_loop(0, n // 2, rb, 0)
        if n % 2:
            one_row(r0 + n - 1, (n - 1 + woff + 0.5) * (1.0 / _SCALE))

    def body(j, carry):
        p = base_p + j
        for i in range(4):
            bn = (i + 2) % 4

            def drain():
                pltpu.make_async_copy(bufs[bn], out_slice(p, 0),
                                      sout[bn]).wait()
            if i < 2:
                pl.when(j > 0)(drain)
            else:
                drain()

            if i < 2:
                pltpu.async_copy(in_slice(p, i + 2), bufs[bn], sin[bn])
            else:
                def refill():
                    pltpu.async_copy(in_slice(p + 1, i - 2), bufs[bn],
                                     sin[bn])
                pl.when(j < _PPW - 1)(refill)

            pltpu.make_async_copy(in_slice(p, i), bufs[i], sin[i]).wait()

            if i == 0:
                build_mh(j)

            for kind, r0, n, mrow, woff in _SEGS[i]:
                if kind == "c":
                    seg_const(bufs[i], r0, n, mrow)
                else:
                    seg_lerp(bufs[i], r0, n, mrow, woff)

            pltpu.async_copy(bufs[i], out_slice(p, i), sout[i])
        return carry

    lax.fori_loop(0, _PPW, body, 0)

    last = base_p + _PPW - 1
    pltpu.make_async_copy(bufs[2], out_slice(last, 2), sout[2]).wait()
    pltpu.make_async_copy(bufs[3], out_slice(last, 3), sout[3]).wait()


@jax.jit
def _run(x, mask):
    B, C, H, W = x.shape
    xp = x.reshape(_NP, _H, _H)
    mp = mask.reshape(_NW, _PPW * _S * _S)
    i0, i1, w0, w1 = _host_tables()

    mesh = plsc.VectorSubcoreMesh(core_axis_name="c", subcore_axis_name="s",
                                  num_cores=_NC, num_subcores=_NS)
    fn = functools.partial(
        pl.kernel,
        out_type=jax.ShapeDtypeStruct((_NP, _H, _H), jnp.float32),
        mesh=mesh,
        compiler_params=pltpu.CompilerParams(needs_layout_passes=False),
        scratch_types=[
            pltpu.VMEM((_CH, _H), jnp.float32),
            pltpu.VMEM((_CH, _H), jnp.float32),
            pltpu.VMEM((_CH, _H), jnp.float32),
            pltpu.VMEM((_CH, _H), jnp.float32),
            pltpu.VMEM((_S * _H,), jnp.float32),
            pltpu.VMEM((_PPW * _S * _S,), jnp.float32),
            pltpu.VMEM((_H,), jnp.int32),
            pltpu.VMEM((_H,), jnp.float32),
        ] + [pltpu.SemaphoreType.DMA] * 8,
    )(_sc_body)
    out = fn(xp, mp, jnp.asarray(i0), jnp.asarray(w1))
    return out.reshape(B, C, H, W)


def kernel(x, mask):
    return _run(x, mask)


# speedup vs baseline: 2.0390x; 1.0002x over previous
import functools

import jax
import jax.numpy as jnp
import numpy as np
from jax import lax
from jax.experimental import pallas as pl
from jax.experimental.pallas import tpu as pltpu
from jax.experimental.pallas import tpu_sc as plsc

_MASK_FRAC = 0.5
_S = 16
_H = 224
_SCALE = _H // _S
_NP = 768
_NC = 2
_NS = 16
_NW = _NC * _NS
_PPW = _NP // _NW
_VL = 16
_VPR = _H // _VL
_CH = 56

_SEGS = {
    0: [("c", 0, 7, 0, 0), ("l", 7, 14, 0, 0), ("l", 21, 14, 1, 0),
        ("l", 35, 14, 2, 0), ("l", 49, 7, 3, 0)],
    1: [("l", 0, 7, 3, 7), ("l", 7, 14, 4, 0), ("l", 21, 14, 5, 0),
        ("l", 35, 14, 6, 0), ("l", 49, 7, 7, 0)],
    2: [("l", 0, 7, 7, 7), ("l", 7, 14, 8, 0), ("l", 21, 14, 9, 0),
        ("l", 35, 14, 10, 0), ("l", 49, 7, 11, 0)],
    3: [("l", 0, 7, 11, 7), ("l", 7, 14, 12, 0), ("l", 21, 14, 13, 0),
        ("l", 35, 14, 14, 0), ("c", 49, 7, 15, 0)],
}


def _host_tables():
    s = (np.arange(_H) + 0.5) / _SCALE - 0.5
    f = np.floor(s)
    w = (s - f).astype(np.float32)
    w[f < 0] = 0.0
    i0 = np.clip(f, 0, _S - 1).astype(np.int32)
    i1 = np.clip(f + 1, 0, _S - 1).astype(np.int32)
    return i0, i1, (1.0 - w), w


def _sc_body(x_hbm, mask_hbm, i0_hbm, w1_hbm, out_hbm,
             b0, b1, b2, b3, mh, mv, i0v, w1v,
             si0, si1, si2, si3, so0, so1, so2, so3):
    bufs = [b0, b1, b2, b3]
    sin = [si0, si1, si2, si3]
    sout = [so0, so1, so2, so3]
    wid = lax.axis_index("s") * _NC + lax.axis_index("c")
    base_p = wid * _PPW

    def in_slice(p, cp):
        return x_hbm.at[p, pl.ds(cp * _CH, _CH)]

    def out_slice(p, cp):
        return out_hbm.at[p, pl.ds(cp * _CH, _CH)]

    zero = jnp.zeros((_VL,), jnp.float32)

    pltpu.async_copy(in_slice(base_p, 0), bufs[0], sin[0])
    pltpu.async_copy(in_slice(base_p, 1), bufs[1], sin[1])

    pltpu.sync_copy(i0_hbm, i0v)
    pltpu.sync_copy(w1_hbm, w1v)
    pltpu.sync_copy(mask_hbm.at[wid], mv)

    i0s = [i0v[pl.ds(v * _VL, _VL)] for v in range(_VPR)]
    i1s = [jnp.minimum(i0s[v] + 1, _S - 1) for v in range(_VPR)]
    w1s = [w1v[pl.ds(v * _VL, _VL)] for v in range(_VPR)]
    w0s = [1.0 - w1s[v] for v in range(_VPR)]

    def build_mh(j):
        mbase = j * (_S * _S)

        def r_body(r, c):
            ro = mbase + r * _S
            for v in range(_VPR):
                g0 = plsc.load_gather(mv, [i0s[v] + ro])
                g1 = plsc.load_gather(mv, [i1s[v] + ro])
                mh[pl.ds(r * _H + v * _VL, _VL)] = g0 * w0s[v] + g1 * w1s[v]
            return c
        lax.fori_loop(0, _S, r_body, 0)

    def seg_const(buf, r0, n, mrow):
        sel = [mh[pl.ds(mrow * _H + v * _VL, _VL)] < _MASK_FRAC
               for v in range(_VPR)]

        def one_row(r):
            for v in range(_VPR):
                sl = pl.ds(v * _VL, _VL)
                buf[r, sl] = jnp.where(sel[v], zero, buf[r, sl])

        def rb(i, c):
            r = r0 + 2 * i
            one_row(r)
            one_row(r + 1)
            return c
        lax.fori_loop(0, n // 2, rb, 0)
        if n % 2:
            one_row(r0 + n - 1)

    def seg_lerp(buf, r0, n, mrow, woff):
        m0s = [mh[pl.ds(mrow * _H + v * _VL, _VL)] for v in range(_VPR)]
        dvs = [mh[pl.ds((mrow + 1) * _H + v * _VL, _VL)] - m0s[v]
               for v in range(_VPR)]

        def one_row(r, wy):
            wyv = jnp.full((_VL,), 0.0, jnp.float32) + wy
            for v in range(_VPR):
                sl = pl.ds(v * _VL, _VL)
                m = m0s[v] + wyv * dvs[v]
                buf[r, sl] = jnp.where(m < _MASK_FRAC, zero, buf[r, sl])

        def rb(i, c):
            i2 = 2 * i
            wy = (i2.astype(jnp.float32) + (woff + 0.5)) * (1.0 / _SCALE)
            r = r0 + i2
            one_row(r, wy)
            one_row(r + 1, wy + 1.0 / _SCALE)
            return c
        lax.fori_loop(0, n // 2, rb, 0)
        if n % 2:
            one_row(r0 + n - 1, (n - 1 + woff + 0.5) * (1.0 / _SCALE))

    def body(j, carry):
        p = base_p + j
        for i in range(4):
            bn = (i + 2) % 4

            def drain():
                pltpu.make_async_copy(bufs[bn], out_slice(p, 0),
                                      sout[bn]).wait()
            if i < 2:
                pl.when(j > 0)(drain)
            else:
                drain()

            if i < 2:
                pltpu.async_copy(in_slice(p, i + 2), bufs[bn], sin[bn])
            else:
                def refill():
                    pltpu.async_copy(in_slice(p + 1, i - 2), bufs[bn],
                                     sin[bn])
                pl.when(j < _PPW - 1)(refill)

            pltpu.make_async_copy(in_slice(p, i), bufs[i], sin[i]).wait()

            if i == 0:
                build_mh(j)

            for kind, r0, n, mrow, woff in _SEGS[i]:
                if kind == "c":
                    seg_const(bufs[i], r0, n, mrow)
                else:
                    seg_lerp(bufs[i], r0, n, mrow, woff)

            pltpu.async_copy(bufs[i], out_slice(p, i), sout[i])
        return carry

    lax.fori_loop(0, _PPW, body, 0)

    last = base_p + _PPW - 1
    pltpu.make_async_copy(bufs[2], out_slice(last, 2), sout[2]).wait()
    pltpu.make_async_copy(bufs[3], out_slice(last, 3), sout[3]).wait()


@jax.jit
def _run(x, mask):
    B, C, H, W = x.shape
    xp = x.reshape(_NP, _H, _H)
    mp = mask.reshape(_NW, _PPW * _S * _S)
    i0, i1, w0, w1 = _host_tables()

    mesh = plsc.VectorSubcoreMesh(core_axis_name="c", subcore_axis_name="s",
                                  num_cores=_NC, num_subcores=_NS)
    fn = functools.partial(
        pl.kernel,
        out_type=jax.ShapeDtypeStruct((_NP, _H, _H), jnp.float32),
        mesh=mesh,
        compiler_params=pltpu.CompilerParams(needs_layout_passes=False),
        scratch_types=[
            pltpu.VMEM((_CH, _H), jnp.float32),
            pltpu.VMEM((_CH, _H), jnp.float32),
            pltpu.VMEM((_CH, _H), jnp.float32),
            pltpu.VMEM((_CH, _H), jnp.float32),
            pltpu.VMEM((_S * _H,), jnp.float32),
            pltpu.VMEM((_PPW * _S * _S,), jnp.float32),
            pltpu.VMEM((_H,), jnp.int32),
            pltpu.VMEM((_H,), jnp.float32),
        ] + [pltpu.SemaphoreType.DMA] * 8,
    )(_sc_body)
    out = fn(xp, mp, jnp.asarray(i0), jnp.asarray(w1))
    return out.reshape(B, C, H, W)


def kernel(x, mask):
    return _run(x, mask)
